# baseline (speedup 1.0000x reference)
import jax
import jax.numpy as jnp
from jax import lax
from jax.experimental import pallas as pl
from jax.experimental.pallas import tpu as pltpu
from jax.experimental.pallas import tpu_sc as plsc

_ROWS = 4880
_DIM = 128
_TOTAL = _ROWS * _DIM


def _copy_body(src_hbm, out_hbm, buf):
    pltpu.sync_copy(src_hbm, buf)
    pltpu.sync_copy(buf, out_hbm)


@jax.jit
def kernel(table):
    flat = table.reshape(_TOTAL)
    mesh = plsc.ScalarSubcoreMesh(axis_name="c", num_cores=1)
    out = pl.kernel(
        _copy_body,
        out_type=jax.ShapeDtypeStruct((_TOTAL,), jnp.float32),
        scratch_types=[pltpu.VMEM_SHARED((_TOTAL,), jnp.float32)],
        mesh=mesh,
    )(flat)
    return out.reshape(_ROWS, _DIM)


# speedup vs baseline: 1.0657x; 1.0657x over previous
import jax
import jax.numpy as jnp
from jax import lax
from jax.experimental import pallas as pl
from jax.experimental.pallas import tpu as pltpu
from jax.experimental.pallas import tpu_sc as plsc

_ROWS = 4880
_DIM = 128
_TOTAL = _ROWS * _DIM
_NUM_SUBCORES = 16
_CHUNK = _TOTAL // _NUM_SUBCORES


def _copy_body(src_hbm, out_hbm, buf):
    wid = lax.axis_index("s")
    base = wid * _CHUNK
    pltpu.sync_copy(src_hbm.at[pl.ds(base, _CHUNK)], buf)
    pltpu.sync_copy(buf, out_hbm.at[pl.ds(base, _CHUNK)])


@jax.jit
def kernel(table):
    flat = table.reshape(_TOTAL)
    mesh = plsc.VectorSubcoreMesh(
        core_axis_name="c", subcore_axis_name="s", num_cores=1)
    out = pl.kernel(
        _copy_body,
        out_type=jax.ShapeDtypeStruct((_TOTAL,), jnp.float32),
        scratch_types=[pltpu.VMEM((_CHUNK,), jnp.float32)],
        mesh=mesh,
    )(flat)
    return out.reshape(_ROWS, _DIM)
